# Initial kernel scaffold; baseline (speedup 1.0000x reference)
import jax
import jax.numpy as jnp
from jax.experimental import pallas as pl


def kernel(x, edge_index, params):
    raise NotImplementedError("write your pallas kernel here")


# speedup vs baseline: 45.5336x; 45.5336x over previous
import functools

import jax
import jax.numpy as jnp
from jax import lax
from jax.experimental import pallas as pl
from jax.experimental.pallas import tpu as pltpu
from jax.experimental.pallas import tpu_sc as plsc

F32 = jnp.float32
I32 = jnp.int32
EPS = 1e-5

N = 10000
E = 640000
NC, NS = 2, 16
NW = NC * NS
EPW = E // NW
CH = 2000
NPAD = 10240
STRIPE = NPAD // NS
ET = 8000
EG = E // ET

_MESH = plsc.VectorSubcoreMesh(core_axis_name="c", subcore_axis_name="s")
_SC_PARAMS = pltpu.CompilerParams(use_tc_tiling_on_sc=False,
                                  needs_layout_passes=False)


def _bn(z, g, b):
    mu = jnp.mean(z, axis=0, keepdims=True)
    var = jnp.mean((z - mu) * (z - mu), axis=0, keepdims=True)
    return g * (z - mu) * lax.rsqrt(var + EPS) + b


def _mlp_full(h, p):
    for i in range(3):
        h = jnp.dot(h, p['W%d' % i], preferred_element_type=F32) + p['b%d' % i]
        if i < 2:
            h = jax.nn.relu(_bn(h, p['g%d' % i], p['be%d' % i]))
    return h


def _node_pre_body(x_ref, W0, b0, g0, be0, W1, b1, g1, be1, W2, b2, cW,
                   A_ref, NB_ref, h1_ref):
    x = x_ref[...]
    p = {'W0': W0[...], 'b0': b0[...], 'g0': g0[...], 'be0': be0[...],
         'W1': W1[...], 'b1': b1[...], 'g1': g1[...], 'be1': be1[...],
         'W2': W2[...], 'b2': b2[...]}
    disp = _mlp_full(x[:, 0:4], p)
    zero13 = jnp.zeros((N, 13), F32)
    A_ref[...] = jnp.concatenate([x[:, 0:3] + disp, zero13], axis=1)
    NB_ref[...] = jnp.concatenate(
        [-x[:, 0:3], x[:, 3:4], jnp.zeros((N, 12), F32)], axis=1)
    h1 = jnp.dot(x[:, 0:7], cW[...], preferred_element_type=F32)
    h1_ref[...] = jnp.concatenate([h1, jnp.zeros((N, 9), F32)], axis=1)


def _node_pre(x, p1, cW):
    outs = (jax.ShapeDtypeStruct((N, 16), F32),
            jax.ShapeDtypeStruct((N, 16), F32),
            jax.ShapeDtypeStruct((N, 16), F32))
    args = [x] + [p1[k] for k in
                  ('W0', 'b0', 'g0', 'be0', 'W1', 'b1', 'g1', 'be1', 'W2', 'b2')] + [cW]
    return pl.pallas_call(_node_pre_body, out_shape=outs)(*args)


def _reg_gather_body(A_hbm, NB_hbm, src_hbm, dst_hbm, out_hbm,
                     sidx, didx, bufA, bufB, semA, semB):
    cid = lax.axis_index("c")
    sid = lax.axis_index("s")
    wid = sid * NC + cid
    base = wid * EPW

    def chunk(k, carry):
        off = base + k * CH
        pltpu.sync_copy(src_hbm.at[pl.ds(off, CH)], sidx)
        pltpu.sync_copy(dst_hbm.at[pl.ds(off, CH)], didx)
        cpA = pltpu.async_copy(A_hbm.at[sidx], bufA, semA)
        cpB = pltpu.async_copy(NB_hbm.at[didx], bufB, semB)
        cpA.wait()
        cpB.wait()

        def vec(i, c):
            bufA[i, :] = bufA[i, :] + bufB[i, :]
            return c
        lax.fori_loop(0, CH, vec, 0, unroll=8)
        pltpu.sync_copy(bufA, out_hbm.at[pl.ds(off, CH)])
        return carry
    lax.fori_loop(0, EPW // CH, chunk, 0)


def _reg_gather(A, NB, src, dst):
    f = pl.kernel(
        _reg_gather_body,
        out_type=jax.ShapeDtypeStruct((E, 16), F32),
        mesh=_MESH,
        compiler_params=_SC_PARAMS,
        scratch_types=[
            pltpu.VMEM((CH,), I32),
            pltpu.VMEM((CH,), I32),
            pltpu.VMEM((CH, 16), F32),
            pltpu.VMEM((CH, 16), F32),
            pltpu.SemaphoreType.DMA,
            pltpu.SemaphoreType.DMA,
        ])
    return f(A, NB, src, dst)


def _stats_accum(i, z, s_ref, out_ref):
    @pl.when(i == 0)
    def _():
        s_ref[...] = jnp.zeros_like(s_ref)
    s_ref[0:1, :] += jnp.sum(z, axis=0, keepdims=True)
    s_ref[1:2, :] += jnp.sum(z * z, axis=0, keepdims=True)

    @pl.when(i == EG - 1)
    def _():
        out_ref[...] = s_ref[...]


def _epass1_body(reg_ref, W0, b0, out_ref, s_ref):
    i = pl.program_id(0)
    z1 = jnp.dot(reg_ref[:, 0:4], W0[...], preferred_element_type=F32) + b0[...]
    _stats_accum(i, z1, s_ref, out_ref)


def _epass2_body(reg_ref, W0, b0, aff1, W1, b1, out_ref, s_ref):
    i = pl.program_id(0)
    z1 = jnp.dot(reg_ref[:, 0:4], W0[...], preferred_element_type=F32) + b0[...]
    a1 = jax.nn.relu(z1 * aff1[0:1, :] + aff1[1:2, :])
    z2 = jnp.dot(a1, W1[...], preferred_element_type=F32) + b1[...]
    _stats_accum(i, z2, s_ref, out_ref)


def _epass3_body(reg_ref, W0, b0, aff1, W1, b1, aff2, W2, b2, out_ref):
    z1 = jnp.dot(reg_ref[:, 0:4], W0[...], preferred_element_type=F32) + b0[...]
    a1 = jax.nn.relu(z1 * aff1[0:1, :] + aff1[1:2, :])
    z2 = jnp.dot(a1, W1[...], preferred_element_type=F32) + b1[...]
    a2 = jax.nn.relu(z2 * aff2[0:1, :] + aff2[1:2, :])
    z3 = jnp.dot(a2, W2[...], preferred_element_type=F32) + b2[...]
    out_ref[...] = jax.nn.relu(z3)


def _wspec(shape):
    nd = len(shape)
    return pl.BlockSpec(shape, lambda i: (0,) * nd)


def _edge_pass1(reg, W0, b0):
    return pl.pallas_call(
        _epass1_body,
        grid=(EG,),
        in_specs=[pl.BlockSpec((ET, 16), lambda i: (i, 0)),
                  _wspec((4, 64)), _wspec((1, 64))],
        out_specs=pl.BlockSpec((2, 64), lambda i: (0, 0)),
        out_shape=jax.ShapeDtypeStruct((2, 64), F32),
        scratch_shapes=[pltpu.VMEM((2, 64), F32)],
    )(reg, W0, b0)


def _edge_pass2(reg, W0, b0, aff1, W1, b1):
    return pl.pallas_call(
        _epass2_body,
        grid=(EG,),
        in_specs=[pl.BlockSpec((ET, 16), lambda i: (i, 0)),
                  _wspec((4, 64)), _wspec((1, 64)), _wspec((2, 64)),
                  _wspec((64, 64)), _wspec((1, 64))],
        out_specs=pl.BlockSpec((2, 64), lambda i: (0, 0)),
        out_shape=jax.ShapeDtypeStruct((2, 64), F32),
        scratch_shapes=[pltpu.VMEM((2, 64), F32)],
    )(reg, W0, b0, aff1, W1, b1)


def _edge_pass3(reg, W0, b0, aff1, W1, b1, aff2, W2, b2):
    return pl.pallas_call(
        _epass3_body,
        grid=(EG,),
        in_specs=[pl.BlockSpec((ET, 16), lambda i: (i, 0)),
                  _wspec((4, 64)), _wspec((1, 64)), _wspec((2, 64)),
                  _wspec((64, 64)), _wspec((1, 64)), _wspec((2, 64)),
                  _wspec((64, 1)), _wspec((1, 1))],
        out_specs=pl.BlockSpec((ET, 1), lambda i: (i, 0)),
        out_shape=jax.ShapeDtypeStruct((E, 1), F32),
    )(reg, W0, b0, aff1, W1, b1, aff2, W2, b2)


def _bn_affine(stats, g, be):
    mean = stats[0] / float(E)
    var = stats[1] / float(E) - mean * mean
    s = g * lax.rsqrt(var + EPS)
    return jnp.stack([s, be - mean * s])


def _deg_body(ew_hbm, dst_hbm, z_hbm, out_hbm, didx, ewv, stg, acc):
    cid = lax.axis_index("c")
    sid = lax.axis_index("s")
    wid = sid * NC + cid
    base = wid * EPW
    pltpu.sync_copy(z_hbm, acc.at[pl.ds(sid * STRIPE, STRIPE)])
    plsc.subcore_barrier()

    def chunk(k, carry):
        off = base + k * CH
        pltpu.sync_copy(dst_hbm.at[pl.ds(off, CH)], didx)
        pltpu.sync_copy(ew_hbm.at[pl.ds(off, CH)], ewv)
        pltpu.sync_copy(ewv, acc.at[didx], add=True)
        return carry
    lax.fori_loop(0, EPW // CH, chunk, 0)
    plsc.subcore_barrier()
    pltpu.sync_copy(acc.at[pl.ds(sid * STRIPE, STRIPE)], stg)
    pltpu.sync_copy(stg, out_hbm.at[cid].at[pl.ds(sid * STRIPE, STRIPE)])


def _deg_scatter(ew, dst, zrows1):
    f = pl.kernel(
        _deg_body,
        out_type=jax.ShapeDtypeStruct((NC, NPAD), F32),
        mesh=_MESH,
        compiler_params=_SC_PARAMS,
        scratch_types=[
            pltpu.VMEM((CH,), I32),
            pltpu.VMEM((CH,), F32),
            pltpu.VMEM((STRIPE,), F32),
            pltpu.VMEM_SHARED((NPAD,), F32),
        ])
    return f(ew, dst, zrows1)


def _conv_body(G_hbm, src_hbm, dst_hbm, ew_hbm, z_hbm, out_hbm,
               sidx, didx, ewv, buf, stg, acc, semg):
    cid = lax.axis_index("c")
    sid = lax.axis_index("s")
    wid = sid * NC + cid
    base = wid * EPW
    pltpu.sync_copy(z_hbm, acc.at[pl.ds(sid * STRIPE, STRIPE)])
    plsc.subcore_barrier()

    def chunk(k, carry):
        off = base + k * CH
        pltpu.sync_copy(src_hbm.at[pl.ds(off, CH)], sidx)
        pltpu.sync_copy(dst_hbm.at[pl.ds(off, CH)], didx)
        pltpu.sync_copy(ew_hbm.at[pl.ds(off, CH)], ewv)
        pltpu.async_copy(G_hbm.at[sidx], buf, semg).wait()

        def grp(g, c):
            for j in range(16):
                e = g * 16 + j
                s16 = plsc.load_gather(ewv, [jnp.full((16,), e, I32)])
                buf[e, :] = buf[e, :] * s16
            return c
        lax.fori_loop(0, CH // 16, grp, 0)
        pltpu.sync_copy(buf, acc.at[didx], add=True)
        return carry
    lax.fori_loop(0, EPW // CH, chunk, 0)
    plsc.subcore_barrier()
    pltpu.sync_copy(acc.at[pl.ds(sid * STRIPE, STRIPE)], stg)
    pltpu.sync_copy(stg, out_hbm.at[cid].at[pl.ds(sid * STRIPE, STRIPE)])


def _conv_aggregate(G, src, dst, ew, zrows):
    f = pl.kernel(
        _conv_body,
        out_type=jax.ShapeDtypeStruct((NC, NPAD, 16), F32),
        mesh=_MESH,
        compiler_params=_SC_PARAMS,
        scratch_types=[
            pltpu.VMEM((CH,), I32),
            pltpu.VMEM((CH,), I32),
            pltpu.VMEM((CH,), F32),
            pltpu.VMEM((CH, 16), F32),
            pltpu.VMEM((STRIPE, 16), F32),
            pltpu.VMEM_SHARED((NPAD, 16), F32),
            pltpu.SemaphoreType.DMA,
        ])
    return f(G, src, dst, ew, zrows)


def _mid1_body(degp_ref, h1_ref, dinv_ref, G1_ref):
    deg = 1.0 + degp_ref[0, 0:N, :] + degp_ref[1, 0:N, :]
    dinv = lax.rsqrt(deg)
    dinv_ref[...] = dinv
    G1_ref[...] = dinv * h1_ref[...]


def _mid1(degp, h1):
    outs = (jax.ShapeDtypeStruct((N, 1), F32),
            jax.ShapeDtypeStruct((N, 16), F32))
    return pl.pallas_call(_mid1_body, out_shape=outs)(degp, h1)


def _mid2_body(acc_ref, G1_ref, dinv_ref, b1c,
               W0, b0, g0, be0, W1, b1, g1, be1, W2, b2,
               bng1, bnb1, bng2, bnb2, cW2, G2_ref):
    dinv = dinv_ref[...]
    accsum = acc_ref[0, 0:N, :] + acc_ref[1, 0:N, :]
    out1 = (dinv * (accsum + G1_ref[...]))[:, 0:7] + b1c[...]
    h = _bn(out1, bng1[...], bnb1[...])
    p = {'W0': W0[...], 'b0': b0[...], 'g0': g0[...], 'be0': be0[...],
         'W1': W1[...], 'b1': b1[...], 'g1': g1[...], 'be1': be1[...],
         'W2': W2[...], 'b2': b2[...]}
    h = _mlp_full(h, p)
    h = _bn(jax.nn.relu(h), bng2[...], bnb2[...])
    h2 = jnp.dot(h, cW2[...], preferred_element_type=F32)
    G2_ref[...] = dinv * h2


def _mid2(acc1, G1, dinv, params):
    p = params['mlp1_2']
    args = [acc1, G1, dinv, params['conv1_1_b'].reshape(1, 7),
            p['W0'], p['b0'].reshape(1, 64), p['g0'].reshape(1, 64), p['be0'].reshape(1, 64),
            p['W1'], p['b1'].reshape(1, 64), p['g1'].reshape(1, 64), p['be1'].reshape(1, 64),
            p['W2'], p['b2'].reshape(1, 16),
            params['bn1_1_g'].reshape(1, 7), params['bn1_1_b'].reshape(1, 7),
            params['bn1_2_g'].reshape(1, 16), params['bn1_2_b'].reshape(1, 16),
            params['conv1_2_W']]
    return pl.pallas_call(
        _mid2_body, out_shape=jax.ShapeDtypeStruct((N, 16), F32))(*args)


def _final_body(acc_ref, G2_ref, dinv_ref, b2c,
                W0, b0, g0, be0, W1, b1, g1, be1, W2, b2,
                bng3, bnb3, bng4, bnb4, linW, linb, out_ref):
    dinv = dinv_ref[...]
    accsum = acc_ref[0, 0:N, :] + acc_ref[1, 0:N, :]
    out2 = dinv * (accsum + G2_ref[...]) + b2c[...]
    h = _bn(out2, bng3[...], bnb3[...])
    p = {'W0': W0[...], 'b0': b0[...], 'g0': g0[...], 'be0': be0[...],
         'W1': W1[...], 'b1': b1[...], 'g1': g1[...], 'be1': be1[...],
         'W2': W2[...], 'b2': b2[...]}
    h = _mlp_full(h, p)
    h = _bn(jax.nn.relu(h), bng4[...], bnb4[...])
    z = jnp.dot(h, linW[...], preferred_element_type=F32) + linb[...]
    out_ref[...] = jax.nn.sigmoid(z)


def _final(acc2, G2, dinv, params):
    p = params['mlp1_3']
    args = [acc2, G2, dinv, params['conv1_2_b'].reshape(1, 16),
            p['W0'], p['b0'].reshape(1, 64), p['g0'].reshape(1, 64), p['be0'].reshape(1, 64),
            p['W1'], p['b1'].reshape(1, 64), p['g1'].reshape(1, 64), p['be1'].reshape(1, 64),
            p['W2'], p['b2'].reshape(1, 32),
            params['bn1_3_g'].reshape(1, 16), params['bn1_3_b'].reshape(1, 16),
            params['bn1_4_g'].reshape(1, 32), params['bn1_4_b'].reshape(1, 32),
            params['lin_W'], params['lin_b'].reshape(1, 1)]
    return pl.pallas_call(
        _final_body, out_shape=jax.ShapeDtypeStruct((N, 1), F32))(*args)


def kernel(x, edge_index, params):
    src = edge_index[0]
    dst = edge_index[1]
    p4 = params['mlp1_4']

    A, NB, h1 = _node_pre(x, params['mlp1_1'], params['conv1_1_W'])
    reg = _reg_gather(A, NB, src, dst)

    W0, W1, W2 = p4['W0'], p4['W1'], p4['W2']
    b0 = p4['b0'].reshape(1, 64)
    b1 = p4['b1'].reshape(1, 64)
    b2 = p4['b2'].reshape(1, 1)
    st1 = _edge_pass1(reg, W0, b0)
    aff1 = _bn_affine(st1, p4['g0'], p4['be0'])
    st2 = _edge_pass2(reg, W0, b0, aff1, W1, b1)
    aff2 = _bn_affine(st2, p4['g1'], p4['be1'])
    ew2 = _edge_pass3(reg, W0, b0, aff1, W1, b1, aff2, W2, b2)
    ew = ew2.reshape(E)

    zrows1 = jnp.zeros((STRIPE,), F32)
    zrows16 = jnp.zeros((STRIPE, 16), F32)
    degp = _deg_scatter(ew, dst, zrows1)
    dinv, G1 = _mid1(degp.reshape(NC, NPAD, 1), h1)

    acc1 = _conv_aggregate(G1, src, dst, ew, zrows16)
    G2 = _mid2(acc1, G1, dinv, params)
    acc2 = _conv_aggregate(G2, src, dst, ew, zrows16)
    return _final(acc2, G2, dinv, params)


# speedup vs baseline: 54.5537x; 1.1981x over previous
import functools

import jax
import jax.numpy as jnp
from jax import lax
from jax.experimental import pallas as pl
from jax.experimental.pallas import tpu as pltpu
from jax.experimental.pallas import tpu_sc as plsc

F32 = jnp.float32
I32 = jnp.int32
EPS = 1e-5

N = 10000
E = 640000
NC, NS = 2, 16
NW = NC * NS
EPW = E // NW
CH = 2000
NPAD = 10240
STRIPE = NPAD // NS
ET = 8000
EG = E // ET

_MESH = plsc.VectorSubcoreMesh(core_axis_name="c", subcore_axis_name="s")
_SC_PARAMS = pltpu.CompilerParams(use_tc_tiling_on_sc=False,
                                  needs_layout_passes=False)


def _bn(z, g, b):
    mu = jnp.mean(z, axis=0, keepdims=True)
    var = jnp.mean((z - mu) * (z - mu), axis=0, keepdims=True)
    return g * (z - mu) * lax.rsqrt(var + EPS) + b


def _mlp_full(h, p):
    for i in range(3):
        h = jnp.dot(h, p['W%d' % i], preferred_element_type=F32) + p['b%d' % i]
        if i < 2:
            h = jax.nn.relu(_bn(h, p['g%d' % i], p['be%d' % i]))
    return h


def _node_pre_body(x_ref, W0, b0, g0, be0, W1, b1, g1, be1, W2, b2, cW,
                   A_ref, NB_ref, h1_ref):
    x = x_ref[...]
    p = {'W0': W0[...], 'b0': b0[...], 'g0': g0[...], 'be0': be0[...],
         'W1': W1[...], 'b1': b1[...], 'g1': g1[...], 'be1': be1[...],
         'W2': W2[...], 'b2': b2[...]}
    disp = _mlp_full(x[:, 0:4], p)
    zero13 = jnp.zeros((N, 13), F32)
    A_ref[...] = jnp.concatenate([x[:, 0:3] + disp, zero13], axis=1)
    NB_ref[...] = jnp.concatenate(
        [-x[:, 0:3], x[:, 3:4], jnp.zeros((N, 12), F32)], axis=1)
    h1 = jnp.dot(x[:, 0:7], cW[...], preferred_element_type=F32)
    h1_ref[...] = jnp.concatenate([h1, jnp.zeros((N, 9), F32)], axis=1)


def _node_pre(x, p1, cW):
    outs = (jax.ShapeDtypeStruct((N, 16), F32),
            jax.ShapeDtypeStruct((N, 16), F32),
            jax.ShapeDtypeStruct((N, 16), F32))
    args = [x] + [p1[k] for k in
                  ('W0', 'b0', 'g0', 'be0', 'W1', 'b1', 'g1', 'be1', 'W2', 'b2')] + [cW]
    return pl.pallas_call(_node_pre_body, out_shape=outs)(*args)


CHR = 1000
NKR = EPW // CHR


def _reg_gather_body(A_hbm, NB_hbm, src_hbm, dst_hbm, out_hbm,
                     sidx0, didx0, bufA0, bufB0, sidx1, didx1, bufA1, bufB1,
                     semA0, semB0, semA1, semB1):
    cid = lax.axis_index("c")
    sid = lax.axis_index("s")
    wid = sid * NC + cid
    base = wid * EPW

    def load_idx(k, sidx, didx):
        off = base + k * CHR
        pltpu.sync_copy(src_hbm.at[pl.ds(off, CHR)], sidx)
        pltpu.sync_copy(dst_hbm.at[pl.ds(off, CHR)], didx)

    def start(sidx, didx, bufA, bufB, semA, semB):
        pltpu.async_copy(A_hbm.at[sidx], bufA, semA)
        pltpu.async_copy(NB_hbm.at[didx], bufB, semB)

    def finish(k, sidx, bufA, bufB, semA, semB):
        pltpu.make_async_copy(A_hbm.at[sidx], bufA, semA).wait()
        pltpu.make_async_copy(A_hbm.at[sidx], bufB, semB).wait()

        def vec(i, c):
            bufA[i, :] = bufA[i, :] + bufB[i, :]
            return c
        lax.fori_loop(0, CHR, vec, 0, unroll=8)
        pltpu.sync_copy(bufA, out_hbm.at[pl.ds(base + k * CHR, CHR)])

    load_idx(0, sidx0, didx0)
    start(sidx0, didx0, bufA0, bufB0, semA0, semB0)

    def pair(t, carry):
        a = 2 * t
        b = a + 1
        load_idx(b, sidx1, didx1)
        start(sidx1, didx1, bufA1, bufB1, semA1, semB1)
        finish(a, sidx0, bufA0, bufB0, semA0, semB0)

        @pl.when(t < NKR // 2 - 1)
        def _():
            load_idx(a + 2, sidx0, didx0)
            start(sidx0, didx0, bufA0, bufB0, semA0, semB0)
        finish(b, sidx1, bufA1, bufB1, semA1, semB1)
        return carry
    lax.fori_loop(0, NKR // 2, pair, 0)


def _reg_gather(A, NB, src, dst):
    f = pl.kernel(
        _reg_gather_body,
        out_type=jax.ShapeDtypeStruct((E, 16), F32),
        mesh=_MESH,
        compiler_params=_SC_PARAMS,
        scratch_types=[
            pltpu.VMEM((CHR,), I32),
            pltpu.VMEM((CHR,), I32),
            pltpu.VMEM((CHR, 16), F32),
            pltpu.VMEM((CHR, 16), F32),
            pltpu.VMEM((CHR,), I32),
            pltpu.VMEM((CHR,), I32),
            pltpu.VMEM((CHR, 16), F32),
            pltpu.VMEM((CHR, 16), F32),
            pltpu.SemaphoreType.DMA,
            pltpu.SemaphoreType.DMA,
            pltpu.SemaphoreType.DMA,
            pltpu.SemaphoreType.DMA,
        ])
    return f(A, NB, src, dst)


def _stats_accum(i, z, s_ref, out_ref):
    @pl.when(i == 0)
    def _():
        s_ref[...] = jnp.zeros_like(s_ref)
    s_ref[0:1, :] += jnp.sum(z, axis=0, keepdims=True)
    s_ref[1:2, :] += jnp.sum(z * z, axis=0, keepdims=True)

    @pl.when(i == EG - 1)
    def _():
        out_ref[...] = s_ref[...]


def _epass1_body(reg_ref, W0, b0, out_ref, s_ref):
    i = pl.program_id(0)
    z1 = jnp.dot(reg_ref[:, 0:4], W0[...], preferred_element_type=F32) + b0[...]
    _stats_accum(i, z1, s_ref, out_ref)


def _epass2_body(reg_ref, W0, b0, aff1, W1, b1, out_ref, s_ref):
    i = pl.program_id(0)
    z1 = jnp.dot(reg_ref[:, 0:4], W0[...], preferred_element_type=F32) + b0[...]
    a1 = jax.nn.relu(z1 * aff1[0:1, :] + aff1[1:2, :])
    z2 = jnp.dot(a1, W1[...], preferred_element_type=F32) + b1[...]
    _stats_accum(i, z2, s_ref, out_ref)


def _epass3_body(reg_ref, W0, b0, aff1, W1, b1, aff2, W2, b2, out_ref):
    z1 = jnp.dot(reg_ref[:, 0:4], W0[...], preferred_element_type=F32) + b0[...]
    a1 = jax.nn.relu(z1 * aff1[0:1, :] + aff1[1:2, :])
    z2 = jnp.dot(a1, W1[...], preferred_element_type=F32) + b1[...]
    a2 = jax.nn.relu(z2 * aff2[0:1, :] + aff2[1:2, :])
    z3 = jnp.dot(a2, W2[...], preferred_element_type=F32) + b2[...]
    out_ref[...] = jax.nn.relu(z3)


def _wspec(shape):
    nd = len(shape)
    return pl.BlockSpec(shape, lambda i: (0,) * nd)


def _edge_pass1(reg, W0, b0):
    return pl.pallas_call(
        _epass1_body,
        grid=(EG,),
        in_specs=[pl.BlockSpec((ET, 16), lambda i: (i, 0)),
                  _wspec((4, 64)), _wspec((1, 64))],
        out_specs=pl.BlockSpec((2, 64), lambda i: (0, 0)),
        out_shape=jax.ShapeDtypeStruct((2, 64), F32),
        scratch_shapes=[pltpu.VMEM((2, 64), F32)],
    )(reg, W0, b0)


def _edge_pass2(reg, W0, b0, aff1, W1, b1):
    return pl.pallas_call(
        _epass2_body,
        grid=(EG,),
        in_specs=[pl.BlockSpec((ET, 16), lambda i: (i, 0)),
                  _wspec((4, 64)), _wspec((1, 64)), _wspec((2, 64)),
                  _wspec((64, 64)), _wspec((1, 64))],
        out_specs=pl.BlockSpec((2, 64), lambda i: (0, 0)),
        out_shape=jax.ShapeDtypeStruct((2, 64), F32),
        scratch_shapes=[pltpu.VMEM((2, 64), F32)],
    )(reg, W0, b0, aff1, W1, b1)


def _edge_pass3(reg, W0, b0, aff1, W1, b1, aff2, W2, b2):
    return pl.pallas_call(
        _epass3_body,
        grid=(EG,),
        in_specs=[pl.BlockSpec((ET, 16), lambda i: (i, 0)),
                  _wspec((4, 64)), _wspec((1, 64)), _wspec((2, 64)),
                  _wspec((64, 64)), _wspec((1, 64)), _wspec((2, 64)),
                  _wspec((64, 1)), _wspec((1, 1))],
        out_specs=pl.BlockSpec((ET, 1), lambda i: (i, 0)),
        out_shape=jax.ShapeDtypeStruct((E, 1), F32),
    )(reg, W0, b0, aff1, W1, b1, aff2, W2, b2)


def _bn_affine(stats, g, be):
    mean = stats[0] / float(E)
    var = stats[1] / float(E) - mean * mean
    s = g * lax.rsqrt(var + EPS)
    return jnp.stack([s, be - mean * s])


def _deg_body(ew_hbm, dst_hbm, z_hbm, out_hbm, didx, ewv, stg, acc):
    cid = lax.axis_index("c")
    sid = lax.axis_index("s")
    wid = sid * NC + cid
    base = wid * EPW
    pltpu.sync_copy(z_hbm, acc.at[pl.ds(sid * STRIPE, STRIPE)])
    plsc.subcore_barrier()

    def chunk(k, carry):
        off = base + k * CH
        pltpu.sync_copy(dst_hbm.at[pl.ds(off, CH)], didx)
        pltpu.sync_copy(ew_hbm.at[pl.ds(off, CH)], ewv)
        pltpu.sync_copy(ewv, acc.at[didx], add=True)
        return carry
    lax.fori_loop(0, EPW // CH, chunk, 0)
    plsc.subcore_barrier()
    pltpu.sync_copy(acc.at[pl.ds(sid * STRIPE, STRIPE)], stg)
    pltpu.sync_copy(stg, out_hbm.at[cid].at[pl.ds(sid * STRIPE, STRIPE)])


def _deg_scatter(ew, dst, zrows1):
    f = pl.kernel(
        _deg_body,
        out_type=jax.ShapeDtypeStruct((NC, NPAD), F32),
        mesh=_MESH,
        compiler_params=_SC_PARAMS,
        scratch_types=[
            pltpu.VMEM((CH,), I32),
            pltpu.VMEM((CH,), F32),
            pltpu.VMEM((STRIPE,), F32),
            pltpu.VMEM_SHARED((NPAD,), F32),
        ])
    return f(ew, dst, zrows1)


_GDN = lax.GatherDimensionNumbers(
    offset_dims=(), collapsed_slice_dims=(0,), start_index_map=(0,))


def _vsplat(v, j):
    idx = jnp.full((16, 1), j, I32)
    return lax.gather(v, idx, _GDN, (1,),
                      mode=lax.GatherScatterMode.PROMISE_IN_BOUNDS)


def _conv_body(G_hbm, src_hbm, dst_hbm, ew_hbm, z_hbm, out_hbm,
               sidx0, didx0, ewv0, buf0, sidx1, didx1, ewv1, buf1,
               stg, acc, sem0, sem1):
    cid = lax.axis_index("c")
    sid = lax.axis_index("s")
    wid = sid * NC + cid
    base = wid * EPW
    pltpu.sync_copy(z_hbm, acc.at[pl.ds(sid * STRIPE, STRIPE)])
    plsc.subcore_barrier()

    def load_idx(k, sidx, didx, ewv):
        off = base + k * CH
        pltpu.sync_copy(src_hbm.at[pl.ds(off, CH)], sidx)
        pltpu.sync_copy(dst_hbm.at[pl.ds(off, CH)], didx)
        pltpu.sync_copy(ew_hbm.at[pl.ds(off, CH)], ewv)

    def finish(sidx, didx, ewv, buf, sem):
        pltpu.make_async_copy(G_hbm.at[sidx], buf, sem).wait()

        def grp(g, c):
            ew16 = ewv[pl.ds(g * 16, 16)]
            for j in range(16):
                e = g * 16 + j
                buf[e, :] = buf[e, :] * _vsplat(ew16, j)
            return c
        lax.fori_loop(0, CH // 16, grp, 0)
        pltpu.sync_copy(buf, acc.at[didx], add=True)

    NK = EPW // CH
    load_idx(0, sidx0, didx0, ewv0)
    pltpu.async_copy(G_hbm.at[sidx0], buf0, sem0)

    def pair(t, carry):
        a = 2 * t
        b = a + 1
        load_idx(b, sidx1, didx1, ewv1)
        pltpu.async_copy(G_hbm.at[sidx1], buf1, sem1)
        finish(sidx0, didx0, ewv0, buf0, sem0)

        @pl.when(t < NK // 2 - 1)
        def _():
            load_idx(a + 2, sidx0, didx0, ewv0)
            pltpu.async_copy(G_hbm.at[sidx0], buf0, sem0)
        finish(sidx1, didx1, ewv1, buf1, sem1)
        return carry
    lax.fori_loop(0, NK // 2, pair, 0)
    plsc.subcore_barrier()
    pltpu.sync_copy(acc.at[pl.ds(sid * STRIPE, STRIPE)], stg)
    pltpu.sync_copy(stg, out_hbm.at[cid].at[pl.ds(sid * STRIPE, STRIPE)])


def _conv_aggregate(G, src, dst, ew, zrows):
    f = pl.kernel(
        _conv_body,
        out_type=jax.ShapeDtypeStruct((NC, NPAD, 16), F32),
        mesh=_MESH,
        compiler_params=_SC_PARAMS,
        scratch_types=[
            pltpu.VMEM((CH,), I32),
            pltpu.VMEM((CH,), I32),
            pltpu.VMEM((CH,), F32),
            pltpu.VMEM((CH, 16), F32),
            pltpu.VMEM((CH,), I32),
            pltpu.VMEM((CH,), I32),
            pltpu.VMEM((CH,), F32),
            pltpu.VMEM((CH, 16), F32),
            pltpu.VMEM((STRIPE, 16), F32),
            pltpu.VMEM_SHARED((NPAD, 16), F32),
            pltpu.SemaphoreType.DMA,
            pltpu.SemaphoreType.DMA,
        ])
    return f(G, src, dst, ew, zrows)


def _mid1_body(degp_ref, h1_ref, dinv_ref, G1_ref):
    deg = 1.0 + degp_ref[0, 0:N, :] + degp_ref[1, 0:N, :]
    dinv = lax.rsqrt(deg)
    dinv_ref[...] = dinv
    G1_ref[...] = dinv * h1_ref[...]


def _mid1(degp, h1):
    outs = (jax.ShapeDtypeStruct((N, 1), F32),
            jax.ShapeDtypeStruct((N, 16), F32))
    return pl.pallas_call(_mid1_body, out_shape=outs)(degp, h1)


def _mid2_body(acc_ref, G1_ref, dinv_ref, b1c,
               W0, b0, g0, be0, W1, b1, g1, be1, W2, b2,
               bng1, bnb1, bng2, bnb2, cW2, G2_ref):
    dinv = dinv_ref[...]
    accsum = acc_ref[0, 0:N, :] + acc_ref[1, 0:N, :]
    out1 = (dinv * (accsum + G1_ref[...]))[:, 0:7] + b1c[...]
    h = _bn(out1, bng1[...], bnb1[...])
    p = {'W0': W0[...], 'b0': b0[...], 'g0': g0[...], 'be0': be0[...],
         'W1': W1[...], 'b1': b1[...], 'g1': g1[...], 'be1': be1[...],
         'W2': W2[...], 'b2': b2[...]}
    h = _mlp_full(h, p)
    h = _bn(jax.nn.relu(h), bng2[...], bnb2[...])
    h2 = jnp.dot(h, cW2[...], preferred_element_type=F32)
    G2_ref[...] = dinv * h2


def _mid2(acc1, G1, dinv, params):
    p = params['mlp1_2']
    args = [acc1, G1, dinv, params['conv1_1_b'].reshape(1, 7),
            p['W0'], p['b0'].reshape(1, 64), p['g0'].reshape(1, 64), p['be0'].reshape(1, 64),
            p['W1'], p['b1'].reshape(1, 64), p['g1'].reshape(1, 64), p['be1'].reshape(1, 64),
            p['W2'], p['b2'].reshape(1, 16),
            params['bn1_1_g'].reshape(1, 7), params['bn1_1_b'].reshape(1, 7),
            params['bn1_2_g'].reshape(1, 16), params['bn1_2_b'].reshape(1, 16),
            params['conv1_2_W']]
    return pl.pallas_call(
        _mid2_body, out_shape=jax.ShapeDtypeStruct((N, 16), F32))(*args)


def _final_body(acc_ref, G2_ref, dinv_ref, b2c,
                W0, b0, g0, be0, W1, b1, g1, be1, W2, b2,
                bng3, bnb3, bng4, bnb4, linW, linb, out_ref):
    dinv = dinv_ref[...]
    accsum = acc_ref[0, 0:N, :] + acc_ref[1, 0:N, :]
    out2 = dinv * (accsum + G2_ref[...]) + b2c[...]
    h = _bn(out2, bng3[...], bnb3[...])
    p = {'W0': W0[...], 'b0': b0[...], 'g0': g0[...], 'be0': be0[...],
         'W1': W1[...], 'b1': b1[...], 'g1': g1[...], 'be1': be1[...],
         'W2': W2[...], 'b2': b2[...]}
    h = _mlp_full(h, p)
    h = _bn(jax.nn.relu(h), bng4[...], bnb4[...])
    z = jnp.dot(h, linW[...], preferred_element_type=F32) + linb[...]
    out_ref[...] = jax.nn.sigmoid(z)


def _final(acc2, G2, dinv, params):
    p = params['mlp1_3']
    args = [acc2, G2, dinv, params['conv1_2_b'].reshape(1, 16),
            p['W0'], p['b0'].reshape(1, 64), p['g0'].reshape(1, 64), p['be0'].reshape(1, 64),
            p['W1'], p['b1'].reshape(1, 64), p['g1'].reshape(1, 64), p['be1'].reshape(1, 64),
            p['W2'], p['b2'].reshape(1, 32),
            params['bn1_3_g'].reshape(1, 16), params['bn1_3_b'].reshape(1, 16),
            params['bn1_4_g'].reshape(1, 32), params['bn1_4_b'].reshape(1, 32),
            params['lin_W'], params['lin_b'].reshape(1, 1)]
    return pl.pallas_call(
        _final_body, out_shape=jax.ShapeDtypeStruct((N, 1), F32))(*args)


def kernel(x, edge_index, params):
    src = edge_index[0]
    dst = edge_index[1]
    p4 = params['mlp1_4']

    A, NB, h1 = _node_pre(x, params['mlp1_1'], params['conv1_1_W'])
    reg = _reg_gather(A, NB, src, dst)

    W0, W1, W2 = p4['W0'], p4['W1'], p4['W2']
    b0 = p4['b0'].reshape(1, 64)
    b1 = p4['b1'].reshape(1, 64)
    b2 = p4['b2'].reshape(1, 1)
    st1 = _edge_pass1(reg, W0, b0)
    aff1 = _bn_affine(st1, p4['g0'], p4['be0'])
    st2 = _edge_pass2(reg, W0, b0, aff1, W1, b1)
    aff2 = _bn_affine(st2, p4['g1'], p4['be1'])
    ew2 = _edge_pass3(reg, W0, b0, aff1, W1, b1, aff2, W2, b2)
    ew = ew2.reshape(E)

    zrows1 = jnp.zeros((STRIPE,), F32)
    zrows16 = jnp.zeros((STRIPE, 16), F32)
    degp = _deg_scatter(ew, dst, zrows1)
    dinv, G1 = _mid1(degp.reshape(NC, NPAD, 1), h1)

    acc1 = _conv_aggregate(G1, src, dst, ew, zrows16)
    G2 = _mid2(acc1, G1, dinv, params)
    acc2 = _conv_aggregate(G2, src, dst, ew, zrows16)
    return _final(acc2, G2, dinv, params)


# speedup vs baseline: 96.0817x; 1.7612x over previous
import functools

import jax
import jax.numpy as jnp
from jax import lax
from jax.experimental import pallas as pl
from jax.experimental.pallas import tpu as pltpu
from jax.experimental.pallas import tpu_sc as plsc

F32 = jnp.float32
I32 = jnp.int32
EPS = 1e-5

N = 10000
E = 640000
NC, NS = 2, 16
NW = NC * NS
EPW = E // NW
CH = 2000
NPAD = 10240
STRIPE = NPAD // NS
ET = 8000
EG = E // ET

_MESH = plsc.VectorSubcoreMesh(core_axis_name="c", subcore_axis_name="s")
_SC_PARAMS = pltpu.CompilerParams(use_tc_tiling_on_sc=False,
                                  needs_layout_passes=False)


def _bn(z, g, b):
    mu = jnp.mean(z, axis=0, keepdims=True)
    var = jnp.mean((z - mu) * (z - mu), axis=0, keepdims=True)
    return g * (z - mu) * lax.rsqrt(var + EPS) + b


def _mlp_full(h, p):
    for i in range(3):
        h = jnp.dot(h, p['W%d' % i], preferred_element_type=F32) + p['b%d' % i]
        if i < 2:
            h = jax.nn.relu(_bn(h, p['g%d' % i], p['be%d' % i]))
    return h


def _node_pre_body(x_ref, W0, b0, g0, be0, W1, b1, g1, be1, W2, b2, cW,
                   A_ref, NB_ref, h1_ref):
    x = x_ref[...]
    p = {'W0': W0[...], 'b0': b0[...], 'g0': g0[...], 'be0': be0[...],
         'W1': W1[...], 'b1': b1[...], 'g1': g1[...], 'be1': be1[...],
         'W2': W2[...], 'b2': b2[...]}
    disp = _mlp_full(x[:, 0:4], p)
    zero13 = jnp.zeros((N, 13), F32)
    A_ref[...] = jnp.concatenate([x[:, 0:3] + disp, zero13], axis=1)
    NB_ref[...] = jnp.concatenate(
        [-x[:, 0:3], x[:, 3:4], jnp.zeros((N, 12), F32)], axis=1)
    h1 = jnp.dot(x[:, 0:7], cW[...], preferred_element_type=F32)
    h1_ref[...] = jnp.concatenate([h1, jnp.zeros((N, 9), F32)], axis=1)


def _node_pre(x, p1, cW):
    outs = (jax.ShapeDtypeStruct((N, 16), F32),
            jax.ShapeDtypeStruct((N, 16), F32),
            jax.ShapeDtypeStruct((N, 16), F32))
    args = [x] + [p1[k] for k in
                  ('W0', 'b0', 'g0', 'be0', 'W1', 'b1', 'g1', 'be1', 'W2', 'b2')] + [cW]
    return pl.pallas_call(_node_pre_body, out_shape=outs)(*args)


CHR = 1000
NKR = EPW // CHR


def _reg_gather_body(A_hbm, NB_hbm, src_hbm, dst_hbm, out_hbm,
                     sidx0, didx0, bufA0, bufB0, sidx1, didx1, bufA1, bufB1,
                     semA0, semB0, semA1, semB1):
    cid = lax.axis_index("c")
    sid = lax.axis_index("s")
    wid = sid * NC + cid
    base = wid * EPW

    def load_idx(k, sidx, didx):
        off = base + k * CHR
        pltpu.sync_copy(src_hbm.at[pl.ds(off, CHR)], sidx)
        pltpu.sync_copy(dst_hbm.at[pl.ds(off, CHR)], didx)

    def start(sidx, didx, bufA, bufB, semA, semB):
        pltpu.async_copy(A_hbm.at[sidx], bufA, semA)
        pltpu.async_copy(NB_hbm.at[didx], bufB, semB)

    def finish(k, sidx, bufA, bufB, semA, semB):
        pltpu.make_async_copy(A_hbm.at[sidx], bufA, semA).wait()
        pltpu.make_async_copy(A_hbm.at[sidx], bufB, semB).wait()

        def vec(i, c):
            bufA[i, :] = bufA[i, :] + bufB[i, :]
            return c
        lax.fori_loop(0, CHR, vec, 0, unroll=8)
        pltpu.sync_copy(bufA, out_hbm.at[pl.ds(base + k * CHR, CHR)])

    load_idx(0, sidx0, didx0)
    start(sidx0, didx0, bufA0, bufB0, semA0, semB0)

    def pair(t, carry):
        a = 2 * t
        b = a + 1
        load_idx(b, sidx1, didx1)
        start(sidx1, didx1, bufA1, bufB1, semA1, semB1)
        finish(a, sidx0, bufA0, bufB0, semA0, semB0)

        @pl.when(t < NKR // 2 - 1)
        def _():
            load_idx(a + 2, sidx0, didx0)
            start(sidx0, didx0, bufA0, bufB0, semA0, semB0)
        finish(b, sidx1, bufA1, bufB1, semA1, semB1)
        return carry
    lax.fori_loop(0, NKR // 2, pair, 0)


def _reg_gather(A, NB, src, dst):
    f = pl.kernel(
        _reg_gather_body,
        out_type=jax.ShapeDtypeStruct((E, 16), F32),
        mesh=_MESH,
        compiler_params=_SC_PARAMS,
        scratch_types=[
            pltpu.VMEM((CHR,), I32),
            pltpu.VMEM((CHR,), I32),
            pltpu.VMEM((CHR, 16), F32),
            pltpu.VMEM((CHR, 16), F32),
            pltpu.VMEM((CHR,), I32),
            pltpu.VMEM((CHR,), I32),
            pltpu.VMEM((CHR, 16), F32),
            pltpu.VMEM((CHR, 16), F32),
            pltpu.SemaphoreType.DMA,
            pltpu.SemaphoreType.DMA,
            pltpu.SemaphoreType.DMA,
            pltpu.SemaphoreType.DMA,
        ])
    return f(A, NB, src, dst)


RP = E // 8
BR1 = 2000
BR2 = 1000
BR3 = 640


def _gram_body(reg_ref, gram_ref, csum_ref, gs_ref, cs_ref):
    i = pl.program_id(0)
    blk = reg_ref[...]
    g = lax.dot_general(blk, blk, (((0,), (0,)), ((), ())),
                        preferred_element_type=F32)
    s = jnp.sum(blk, axis=0, keepdims=True)

    @pl.when(i == 0)
    def _():
        gs_ref[...] = jnp.zeros_like(gs_ref)
        cs_ref[...] = jnp.zeros_like(cs_ref)
    gs_ref[...] += g
    cs_ref[...] += s

    @pl.when(i == RP // BR1 - 1)
    def _():
        gram_ref[...] = gs_ref[...]
        csum_ref[...] = cs_ref[...]


def _edge_pass1(regp):
    return pl.pallas_call(
        _gram_body,
        grid=(RP // BR1,),
        in_specs=[pl.BlockSpec((BR1, 128), lambda i: (i, 0))],
        out_specs=(pl.BlockSpec((128, 128), lambda i: (0, 0)),
                   pl.BlockSpec((1, 128), lambda i: (0, 0))),
        out_shape=(jax.ShapeDtypeStruct((128, 128), F32),
                   jax.ShapeDtypeStruct((1, 128), F32)),
        scratch_shapes=[pltpu.VMEM((128, 128), F32), pltpu.VMEM((1, 128), F32)],
    )(regp)


def _epass2_body(reg_ref, W0p, b0p, aff1p, gram_ref, csum_ref, gs_ref, cs_ref):
    i = pl.program_id(0)
    z1 = jnp.dot(reg_ref[...], W0p[...], preferred_element_type=F32) + b0p[...]
    a1 = jax.nn.relu(z1 * aff1p[0:1, :] + aff1p[1:2, :])
    g = lax.dot_general(a1, a1, (((0,), (0,)), ((), ())),
                        preferred_element_type=F32)
    s = jnp.sum(a1, axis=0, keepdims=True)

    @pl.when(i == 0)
    def _():
        gs_ref[...] = jnp.zeros_like(gs_ref)
        cs_ref[...] = jnp.zeros_like(cs_ref)
    gs_ref[...] += g
    cs_ref[...] += s

    @pl.when(i == RP // BR2 - 1)
    def _():
        gram_ref[...] = gs_ref[...]
        csum_ref[...] = cs_ref[...]


def _edge_pass2(regp, W0p, b0p, aff1p):
    return pl.pallas_call(
        _epass2_body,
        grid=(RP // BR2,),
        in_specs=[pl.BlockSpec((BR2, 128), lambda i: (i, 0)),
                  _wspec((128, 512)), _wspec((1, 512)), _wspec((2, 512))],
        out_specs=(pl.BlockSpec((512, 512), lambda i: (0, 0)),
                   pl.BlockSpec((1, 512), lambda i: (0, 0))),
        out_shape=(jax.ShapeDtypeStruct((512, 512), F32),
                   jax.ShapeDtypeStruct((1, 512), F32)),
        scratch_shapes=[pltpu.VMEM((512, 512), F32), pltpu.VMEM((1, 512), F32)],
    )(regp, W0p, b0p, aff1p)


def _epass3_body(reg_ref, W0p, b0p, aff1p, W1p, b1p, aff2p, W2p, b2p, out_ref):
    z1 = jnp.dot(reg_ref[...], W0p[...], preferred_element_type=F32) + b0p[...]
    a1 = jax.nn.relu(z1 * aff1p[0:1, :] + aff1p[1:2, :])
    z2 = jnp.dot(a1, W1p[...], preferred_element_type=F32) + b1p[...]
    a2 = jax.nn.relu(z2 * aff2p[0:1, :] + aff2p[1:2, :])
    z3 = jnp.dot(a2, W2p[...], preferred_element_type=F32) + b2p[...]
    out_ref[...] = jax.nn.relu(z3)


def _edge_pass3(regp, W0p, b0p, aff1p, W1p, b1p, aff2p, W2p, b2p):
    return pl.pallas_call(
        _epass3_body,
        grid=(RP // BR3,),
        in_specs=[pl.BlockSpec((BR3, 128), lambda i: (i, 0)),
                  _wspec((128, 512)), _wspec((1, 512)), _wspec((2, 512)),
                  _wspec((512, 512)), _wspec((1, 512)), _wspec((2, 512)),
                  _wspec((512, 8)), _wspec((1, 8))],
        out_specs=pl.BlockSpec((BR3, 8), lambda i: (i, 0)),
        out_shape=jax.ShapeDtypeStruct((RP, 8), F32),
    )(regp, W0p, b0p, aff1p, W1p, b1p, aff2p, W2p, b2p)


def _wspec(shape):
    nd = len(shape)
    return pl.BlockSpec(shape, lambda i: (0,) * nd)


def _affine_from_stats(ssum, ssq, g, be):
    mean = ssum / float(E)
    var = ssq / float(E) - mean * mean
    s = g * lax.rsqrt(var + EPS)
    return jnp.stack([s, be - mean * s])


def _fold_diag(G, k):
    return sum(G[i * k:(i + 1) * k, i * k:(i + 1) * k] for i in range(8))


def _fold_sum(cs, k):
    return cs.reshape(8, k).sum(axis=0)


def _deg_body(ew_hbm, dst_hbm, z_hbm, out_hbm, didx, ewv, stg, acc):
    cid = lax.axis_index("c")
    sid = lax.axis_index("s")
    wid = sid * NC + cid
    base = wid * EPW
    pltpu.sync_copy(z_hbm, acc.at[pl.ds(sid * STRIPE, STRIPE)])
    plsc.subcore_barrier()

    def chunk(k, carry):
        off = base + k * CH
        pltpu.sync_copy(dst_hbm.at[pl.ds(off, CH)], didx)
        pltpu.sync_copy(ew_hbm.at[pl.ds(off, CH)], ewv)
        pltpu.sync_copy(ewv, acc.at[didx], add=True)
        return carry
    lax.fori_loop(0, EPW // CH, chunk, 0)
    plsc.subcore_barrier()
    pltpu.sync_copy(acc.at[pl.ds(sid * STRIPE, STRIPE)], stg)
    pltpu.sync_copy(stg, out_hbm.at[cid].at[pl.ds(sid * STRIPE, STRIPE)])


def _deg_scatter(ew, dst, zrows1):
    f = pl.kernel(
        _deg_body,
        out_type=jax.ShapeDtypeStruct((NC, NPAD), F32),
        mesh=_MESH,
        compiler_params=_SC_PARAMS,
        scratch_types=[
            pltpu.VMEM((CH,), I32),
            pltpu.VMEM((CH,), F32),
            pltpu.VMEM((STRIPE,), F32),
            pltpu.VMEM_SHARED((NPAD,), F32),
        ])
    return f(ew, dst, zrows1)


_GDN = lax.GatherDimensionNumbers(
    offset_dims=(), collapsed_slice_dims=(0,), start_index_map=(0,))


def _vsplat(v, j):
    idx = jnp.full((16, 1), j, I32)
    return lax.gather(v, idx, _GDN, (1,),
                      mode=lax.GatherScatterMode.PROMISE_IN_BOUNDS)


def _conv_body(G_hbm, src_hbm, dst_hbm, ew_hbm, z_hbm, out_hbm,
               sidx0, didx0, ewv0, buf0, sidx1, didx1, ewv1, buf1,
               stg, acc, sem0, sem1):
    cid = lax.axis_index("c")
    sid = lax.axis_index("s")
    wid = sid * NC + cid
    base = wid * EPW
    pltpu.sync_copy(z_hbm, acc.at[pl.ds(sid * STRIPE, STRIPE)])
    plsc.subcore_barrier()

    def load_idx(k, sidx, didx, ewv):
        off = base + k * CH
        pltpu.sync_copy(src_hbm.at[pl.ds(off, CH)], sidx)
        pltpu.sync_copy(dst_hbm.at[pl.ds(off, CH)], didx)
        pltpu.sync_copy(ew_hbm.at[pl.ds(off, CH)], ewv)

    def finish(sidx, didx, ewv, buf, sem):
        pltpu.make_async_copy(G_hbm.at[sidx], buf, sem).wait()

        def grp(g, c):
            ew16 = ewv[pl.ds(g * 16, 16)]
            for j in range(16):
                e = g * 16 + j
                buf[e, :] = buf[e, :] * _vsplat(ew16, j)
            return c
        lax.fori_loop(0, CH // 16, grp, 0)
        pltpu.sync_copy(buf, acc.at[didx], add=True)

    NK = EPW // CH
    load_idx(0, sidx0, didx0, ewv0)
    pltpu.async_copy(G_hbm.at[sidx0], buf0, sem0)

    def pair(t, carry):
        a = 2 * t
        b = a + 1
        load_idx(b, sidx1, didx1, ewv1)
        pltpu.async_copy(G_hbm.at[sidx1], buf1, sem1)
        finish(sidx0, didx0, ewv0, buf0, sem0)

        @pl.when(t < NK // 2 - 1)
        def _():
            load_idx(a + 2, sidx0, didx0, ewv0)
            pltpu.async_copy(G_hbm.at[sidx0], buf0, sem0)
        finish(sidx1, didx1, ewv1, buf1, sem1)
        return carry
    lax.fori_loop(0, NK // 2, pair, 0)
    plsc.subcore_barrier()
    pltpu.sync_copy(acc.at[pl.ds(sid * STRIPE, STRIPE)], stg)
    pltpu.sync_copy(stg, out_hbm.at[cid].at[pl.ds(sid * STRIPE, STRIPE)])


def _conv_aggregate(G, src, dst, ew, zrows):
    f = pl.kernel(
        _conv_body,
        out_type=jax.ShapeDtypeStruct((NC, NPAD, 16), F32),
        mesh=_MESH,
        compiler_params=_SC_PARAMS,
        scratch_types=[
            pltpu.VMEM((CH,), I32),
            pltpu.VMEM((CH,), I32),
            pltpu.VMEM((CH,), F32),
            pltpu.VMEM((CH, 16), F32),
            pltpu.VMEM((CH,), I32),
            pltpu.VMEM((CH,), I32),
            pltpu.VMEM((CH,), F32),
            pltpu.VMEM((CH, 16), F32),
            pltpu.VMEM((STRIPE, 16), F32),
            pltpu.VMEM_SHARED((NPAD, 16), F32),
            pltpu.SemaphoreType.DMA,
            pltpu.SemaphoreType.DMA,
        ])
    return f(G, src, dst, ew, zrows)


def _mid1_body(degp_ref, h1_ref, dinv_ref, G1_ref):
    deg = 1.0 + degp_ref[0, 0:N, :] + degp_ref[1, 0:N, :]
    dinv = lax.rsqrt(deg)
    dinv_ref[...] = dinv
    G1_ref[...] = dinv * h1_ref[...]


def _mid1(degp, h1):
    outs = (jax.ShapeDtypeStruct((N, 1), F32),
            jax.ShapeDtypeStruct((N, 16), F32))
    return pl.pallas_call(_mid1_body, out_shape=outs)(degp, h1)


def _mid2_body(acc_ref, G1_ref, dinv_ref, b1c,
               W0, b0, g0, be0, W1, b1, g1, be1, W2, b2,
               bng1, bnb1, bng2, bnb2, cW2, G2_ref):
    dinv = dinv_ref[...]
    accsum = acc_ref[0, 0:N, :] + acc_ref[1, 0:N, :]
    out1 = (dinv * (accsum + G1_ref[...]))[:, 0:7] + b1c[...]
    h = _bn(out1, bng1[...], bnb1[...])
    p = {'W0': W0[...], 'b0': b0[...], 'g0': g0[...], 'be0': be0[...],
         'W1': W1[...], 'b1': b1[...], 'g1': g1[...], 'be1': be1[...],
         'W2': W2[...], 'b2': b2[...]}
    h = _mlp_full(h, p)
    h = _bn(jax.nn.relu(h), bng2[...], bnb2[...])
    h2 = jnp.dot(h, cW2[...], preferred_element_type=F32)
    G2_ref[...] = dinv * h2


def _mid2(acc1, G1, dinv, params):
    p = params['mlp1_2']
    args = [acc1, G1, dinv, params['conv1_1_b'].reshape(1, 7),
            p['W0'], p['b0'].reshape(1, 64), p['g0'].reshape(1, 64), p['be0'].reshape(1, 64),
            p['W1'], p['b1'].reshape(1, 64), p['g1'].reshape(1, 64), p['be1'].reshape(1, 64),
            p['W2'], p['b2'].reshape(1, 16),
            params['bn1_1_g'].reshape(1, 7), params['bn1_1_b'].reshape(1, 7),
            params['bn1_2_g'].reshape(1, 16), params['bn1_2_b'].reshape(1, 16),
            params['conv1_2_W']]
    return pl.pallas_call(
        _mid2_body, out_shape=jax.ShapeDtypeStruct((N, 16), F32))(*args)


def _final_body(acc_ref, G2_ref, dinv_ref, b2c,
                W0, b0, g0, be0, W1, b1, g1, be1, W2, b2,
                bng3, bnb3, bng4, bnb4, linW, linb, out_ref):
    dinv = dinv_ref[...]
    accsum = acc_ref[0, 0:N, :] + acc_ref[1, 0:N, :]
    out2 = dinv * (accsum + G2_ref[...]) + b2c[...]
    h = _bn(out2, bng3[...], bnb3[...])
    p = {'W0': W0[...], 'b0': b0[...], 'g0': g0[...], 'be0': be0[...],
         'W1': W1[...], 'b1': b1[...], 'g1': g1[...], 'be1': be1[...],
         'W2': W2[...], 'b2': b2[...]}
    h = _mlp_full(h, p)
    h = _bn(jax.nn.relu(h), bng4[...], bnb4[...])
    z = jnp.dot(h, linW[...], preferred_element_type=F32) + linb[...]
    out_ref[...] = jax.nn.sigmoid(z)


def _final(acc2, G2, dinv, params):
    p = params['mlp1_3']
    args = [acc2, G2, dinv, params['conv1_2_b'].reshape(1, 16),
            p['W0'], p['b0'].reshape(1, 64), p['g0'].reshape(1, 64), p['be0'].reshape(1, 64),
            p['W1'], p['b1'].reshape(1, 64), p['g1'].reshape(1, 64), p['be1'].reshape(1, 64),
            p['W2'], p['b2'].reshape(1, 32),
            params['bn1_3_g'].reshape(1, 16), params['bn1_3_b'].reshape(1, 16),
            params['bn1_4_g'].reshape(1, 32), params['bn1_4_b'].reshape(1, 32),
            params['lin_W'], params['lin_b'].reshape(1, 1)]
    return pl.pallas_call(
        _final_body, out_shape=jax.ShapeDtypeStruct((N, 1), F32))(*args)


def kernel(x, edge_index, params):
    src = edge_index[0]
    dst = edge_index[1]
    p4 = params['mlp1_4']

    A, NB, h1 = _node_pre(x, params['mlp1_1'], params['conv1_1_W'])
    reg = _reg_gather(A, NB, src, dst)

    W0 = jnp.concatenate([p4['W0'], jnp.zeros((12, 64), F32)], axis=0)
    W1 = p4['W1']
    eye8 = jnp.eye(8, dtype=F32)
    W0p = jnp.kron(eye8, W0)
    W1p = jnp.kron(eye8, W1)
    W2p = jnp.kron(eye8, p4['W2'])
    b0p = jnp.tile(p4['b0'].reshape(1, 64), (1, 8))
    b1p = jnp.tile(p4['b1'].reshape(1, 64), (1, 8))
    b2p = jnp.tile(p4['b2'].reshape(1, 1), (1, 8))
    regp = reg.reshape(RP, 128)

    gram1, cs1 = _edge_pass1(regp)
    G16 = _fold_diag(gram1, 16)
    c16 = _fold_sum(cs1, 16)
    b0 = p4['b0']
    cW0 = c16 @ W0
    ssum1 = cW0 + float(E) * b0
    ssq1 = ((G16 @ W0) * W0).sum(axis=0) + 2.0 * b0 * cW0 + float(E) * b0 * b0
    aff1 = _affine_from_stats(ssum1, ssq1, p4['g0'], p4['be0'])
    aff1p = jnp.tile(aff1, (1, 8))

    gram2, cs2 = _edge_pass2(regp, W0p, b0p, aff1p)
    G64 = _fold_diag(gram2, 64)
    c64 = _fold_sum(cs2, 64)
    b1 = p4['b1']
    cW1 = c64 @ W1
    ssum2 = cW1 + float(E) * b1
    ssq2 = ((G64 @ W1) * W1).sum(axis=0) + 2.0 * b1 * cW1 + float(E) * b1 * b1
    aff2 = _affine_from_stats(ssum2, ssq2, p4['g1'], p4['be1'])
    aff2p = jnp.tile(aff2, (1, 8))

    ew = _edge_pass3(regp, W0p, b0p, aff1p, W1p, b1p, aff2p, W2p, b2p).reshape(E)

    zrows1 = jnp.zeros((STRIPE,), F32)
    zrows16 = jnp.zeros((STRIPE, 16), F32)
    degp = _deg_scatter(ew, dst, zrows1)
    dinv, G1 = _mid1(degp.reshape(NC, NPAD, 1), h1)

    acc1 = _conv_aggregate(G1, src, dst, ew, zrows16)
    G2 = _mid2(acc1, G1, dinv, params)
    acc2 = _conv_aggregate(G2, src, dst, ew, zrows16)
    return _final(acc2, G2, dinv, params)
